# baseline (speedup 1.0000x reference)
import functools
import jax
import jax.numpy as jnp
from jax import lax
from jax.experimental import pallas as pl
from jax.experimental.pallas import tpu as pltpu
from jax.experimental.pallas import tpu_sc as plsc

D = 128
N0 = 10000
E = 320000
T = 500000
NC, NS = 2, 16
NW = NC * NS

TP = 524288
LW = TP // NW
WS = 2048
RCH = 11776
RG = RCH
NPASS = 14
EPAD = 2 * RCH * NPASS
BB = 64
CB = 2048
KEY_SENTINEL = 1 << 30

_mesh = plsc.VectorSubcoreMesh(core_axis_name="c", subcore_axis_name="s")


def _proj_body(nout, relu0, x_ref, w_ref, b_ref, *out_refs):
    acc = jnp.dot(x_ref[...], w_ref[...], preferred_element_type=jnp.float32)
    for j in range(nout):
        blk = acc[:, j * D:(j + 1) * D]
        if j == 0 and relu0:
            out_refs[j][...] = jnp.maximum(blk + b_ref[0, :], 0.0)
        else:
            out_refs[j][...] = blk


def _proj(x, ws, bq, relu0, bm):
    n = x.shape[0]
    nout = len(ws)
    wcat = jnp.concatenate(ws, axis=1)
    return pl.pallas_call(
        functools.partial(_proj_body, nout, relu0),
        grid=(n // bm,),
        in_specs=[pl.BlockSpec((bm, D), lambda i: (i, 0)),
                  pl.BlockSpec((D, nout * D), lambda i: (0, 0)),
                  pl.BlockSpec((1, D), lambda i: (0, 0))],
        out_specs=[pl.BlockSpec((bm, D), lambda i: (i, 0))] * nout,
        out_shape=[jax.ShapeDtypeStruct((n, D), jnp.float32)] * nout,
    )(x, wcat, bq.reshape(1, D))


def _final12_body(ea_ref, x1, x2, x3, x4, s_ref, w1, w2, w3, w4, wa, wb,
                  bsum, ba, bb_, o_ref):
    acc = jnp.dot(x1[...], w1[...], preferred_element_type=jnp.float32)
    acc += jnp.dot(x2[...], w2[...], preferred_element_type=jnp.float32)
    acc += jnp.dot(x3[...], w3[...], preferred_element_type=jnp.float32)
    acc += jnp.dot(x4[...], w4[...], preferred_element_type=jnp.float32)
    ea = ea_ref[...]
    pre = ea * s_ref[0, :] + acc + bsum[0, :]
    h = jnp.maximum(jnp.dot(pre, wa[...], preferred_element_type=jnp.float32)
                    + ba[0, :], 0.0)
    o_ref[...] = (jnp.dot(h, wb[...], preferred_element_type=jnp.float32)
                  + bb_[0, :] + ea)


def _final12(ea, xs, wsbs, mlp_a, mlp_b, scale_row, bm):
    n = ea.shape[0]
    ws = [p[0] for p in wsbs]
    bsum = sum(p[1] for p in wsbs).reshape(1, D)
    row = pl.BlockSpec((bm, D), lambda i: (i, 0))
    cst = pl.BlockSpec((D, D), lambda i: (0, 0))
    one = pl.BlockSpec((1, D), lambda i: (0, 0))
    return pl.pallas_call(
        _final12_body,
        grid=(n // bm,),
        in_specs=[row, row, row, row, row, one, cst, cst, cst, cst, cst, cst,
                  one, one, one],
        out_specs=row,
        out_shape=jax.ShapeDtypeStruct((n, D), jnp.float32),
    )(ea, xs[0], xs[1], xs[2], xs[3], scale_row, ws[0], ws[1], ws[2], ws[3],
      mlp_a[0], mlp_b[0], bsum, mlp_a[1].reshape(1, D), mlp_b[1].reshape(1, D))


def _final0_body(ea_ref, g1a, g1b, g2a, g2b, s_ref, w1, w2, wa, wb,
                 bsum, ba, bb_, o_ref):
    acc = jnp.dot(g1a[...] + g1b[...], w1[...],
                  preferred_element_type=jnp.float32)
    acc += jnp.dot(g2a[...] + g2b[...], w2[...],
                   preferred_element_type=jnp.float32)
    ea = ea_ref[...]
    pre = ea * s_ref[0, :] + acc + bsum[0, :]
    h = jnp.maximum(jnp.dot(pre, wa[...], preferred_element_type=jnp.float32)
                    + ba[0, :], 0.0)
    o_ref[...] = (jnp.dot(h, wb[...], preferred_element_type=jnp.float32)
                  + bb_[0, :] + ea)


def _final0(ea, g1, g2, p110, p220, mlp_a, mlp_b, scale_row, bm):
    n = ea.shape[0]
    bsum = (p110[1] + p220[1]).reshape(1, D)
    row = pl.BlockSpec((bm, D), lambda i: (i, 0))
    cst = pl.BlockSpec((D, D), lambda i: (0, 0))
    one = pl.BlockSpec((1, D), lambda i: (0, 0))
    return pl.pallas_call(
        _final0_body,
        grid=(n // bm,),
        in_specs=[row, row, row, row, row, one, cst, cst, cst, cst,
                  one, one, one],
        out_specs=row,
        out_shape=jax.ShapeDtypeStruct((n, D), jnp.float32),
    )(ea, g1[0], g1[1], g2[0], g2[1], scale_row, p110[0], p220[0],
      mlp_a[0], mlp_b[0], bsum, mlp_a[1].reshape(1, D), mlp_b[1].reshape(1, D))


_WA = 80
_NWIN_A = (E // NW) // _WA
_WC = 200
_NWIN_C = (E // NW) // _WC


def _edge_stage(p0, s_idx, e_idx, q, bias):
    @functools.partial(
        pl.kernel, mesh=_mesh,
        compiler_params=pltpu.CompilerParams(needs_layout_passes=False),
        out_type=[jax.ShapeDtypeStruct((E, D), jnp.float32),
                  jax.ShapeDtypeStruct((NC, N0, D), jnp.float32)],
        scratch_types=[
            pltpu.VMEM((_WA,), jnp.int32),
            pltpu.VMEM((_WA,), jnp.int32),
            pltpu.VMEM((_WA, D), jnp.float32),
            pltpu.VMEM((_WA, D), jnp.float32),
            pltpu.VMEM((_WA, D), jnp.float32),
            pltpu.VMEM((8, D), jnp.float32),
            pltpu.VMEM((D,), jnp.float32),
            pltpu.VMEM_SHARED((N0, D), jnp.float32),
            pltpu.SemaphoreType.DMA,
            pltpu.SemaphoreType.DMA,
        ],
    )
    def k(p0_h, s_h, e_h, q_h, b_h, h_out, g_out, sv, ev, rs, re, qr,
          zbuf, bv, acc, semA, semB):
        cid = lax.axis_index("c")
        sid = lax.axis_index("s")
        wid = sid * NC + cid
        pltpu.sync_copy(b_h, bv)

        def zrow(r, _):
            for cc in range(8):
                zbuf[r, pl.ds(cc * 16, 16)] = jnp.zeros((16,), jnp.float32)
            return 0
        lax.fori_loop(0, 8, zrow, 0)

        def zslice(z, _):
            pltpu.sync_copy(zbuf, acc.at[pl.ds(sid * 624 + z * 8, 8)])
            return 0
        lax.fori_loop(0, 78, zslice, 0)

        @pl.when(sid == 15)
        def _():
            pltpu.sync_copy(zbuf, acc.at[pl.ds(9984, 8)])
            pltpu.sync_copy(zbuf, acc.at[pl.ds(9992, 8)])
        plsc.subcore_barrier()

        def win(wi, _):
            base = wid * (E // NW) + wi * _WA
            pltpu.sync_copy(s_h.at[pl.ds(base, _WA)], sv)
            pltpu.sync_copy(e_h.at[pl.ds(base, _WA)], ev)
            ca = pltpu.async_copy(p0_h.at[sv], rs, semA)
            cb = pltpu.async_copy(p0_h.at[ev], re, semB)
            pltpu.sync_copy(q_h.at[pl.ds(base, _WA)], qr)
            ca.wait()
            cb.wait()

            def rowfn(r, _):
                for cc in range(8):
                    sl = pl.ds(cc * 16, 16)
                    v = rs[r, sl] + re[r, sl] + bv[sl]
                    rs[r, sl] = jnp.maximum(v, 0.0)
                return 0
            lax.fori_loop(0, _WA, rowfn, 0)
            pltpu.sync_copy(rs, h_out.at[pl.ds(base, _WA)])
            pltpu.sync_copy(qr, acc.at[sv], add=True)
            return 0
        lax.fori_loop(0, _NWIN_A, win, 0)
        plsc.subcore_barrier()
        pltpu.sync_copy(acc.at[pl.ds(sid * 624, 624)],
                        g_out.at[cid, pl.ds(sid * 624, 624)])

        @pl.when(sid == 15)
        def _():
            pltpu.sync_copy(acc.at[pl.ds(9984, 16)],
                            g_out.at[cid, pl.ds(9984, 16)])
    return k(p0, s_idx, e_idx, q, bias)


def _tri_segsum(ta, tb, ia, ib, key, bias):
    nbat = CB // BB

    @functools.partial(
        pl.kernel, mesh=_mesh,
        compiler_params=pltpu.CompilerParams(needs_layout_passes=False),
        out_type=jax.ShapeDtypeStruct((EPAD, D), jnp.float32),
        scratch_types=[
            pltpu.VMEM((WS,), jnp.int32),
            pltpu.VMEM((WS,), jnp.int32),
            pltpu.VMEM((WS,), jnp.int32),
            pltpu.VMEM((nbat, BB), jnp.int32),
            pltpu.VMEM((nbat, BB), jnp.int32),
            pltpu.VMEM((nbat, BB), jnp.int32),
            pltpu.VMEM((BB, D), jnp.float32),
            pltpu.VMEM((BB, D), jnp.float32),
            pltpu.VMEM((8, D), jnp.float32),
            pltpu.VMEM((D,), jnp.float32),
            pltpu.VMEM_SHARED((RCH + 8, D), jnp.float32),
            pltpu.SemaphoreType.DMA,
            pltpu.SemaphoreType.DMA,
        ],
    )
    def k(ta_h, tb_h, ia_h, ib_h, key_h, b_h, s_out, iaw, ibw, keyw,
          ca, cb, ck, ra, rb, zbuf, bv, acc, semA, semB):
        cid = lax.axis_index("c")
        sid = lax.axis_index("s")
        wid = sid * NC + cid
        pltpu.sync_copy(b_h, bv)
        zero16f = jnp.zeros((16,), jnp.float32)
        zero16i = jnp.zeros((16,), jnp.int32)
        lane = lax.iota(jnp.int32, 16)
        dn = lax.GatherDimensionNumbers(
            offset_dims=(), collapsed_slice_dims=(0,), start_index_map=(0,))

        def prefix(mi):
            v = mi
            for off in (1, 2, 4, 8):
                sh = lax.gather(v, jnp.maximum(lane - off, 0)[:, None], dn,
                                slice_sizes=(1,),
                                mode=lax.GatherScatterMode.PROMISE_IN_BOUNDS)
                v = v + jnp.where(lane >= off, sh, 0)
            return v

        def z0(r, _):
            for cc in range(8):
                zbuf[r, pl.ds(cc * 16, 16)] = zero16f
            return 0
        lax.fori_loop(0, 8, z0, 0)

        def z1(bt, _):
            for j in range(BB // 16):
                ca[bt, pl.ds(j * 16, 16)] = zero16i
                cb[bt, pl.ds(j * 16, 16)] = zero16i
                ck[bt, pl.ds(j * 16, 16)] = zero16i + RG
            return 0
        lax.fori_loop(0, nbat, z1, 0)

        def pad_tail(fill):
            for j in range(BB // 16):
                p0 = fill + j * 16 + lane
                mok = p0 < CB
                pos_hi = p0 >> 6
                pos_lo = p0 & (BB - 1)
                plsc.store_scatter(ca, [pos_hi, pos_lo], lane, mask=mok)
                plsc.store_scatter(cb, [pos_hi, pos_lo], lane, mask=mok)
                plsc.store_scatter(ck, [pos_hi, pos_lo], zero16i + RG,
                                   mask=mok)

        def drain_all(fill):
            nb = (fill + BB - 1) // BB

            def dr(b, _):
                g1 = pltpu.async_copy(ta_h.at[ca.at[b]], ra, semA)
                g2 = pltpu.async_copy(tb_h.at[cb.at[b]], rb, semB)
                g1.wait()
                g2.wait()

                def rowfn(r, _):
                    for cc in range(8):
                        sl = pl.ds(cc * 16, 16)
                        ra[r, sl] = jnp.maximum(
                            ra[r, sl] + rb[r, sl] + bv[sl], 0.0)
                    return 0
                lax.fori_loop(0, BB, rowfn, 0)
                pltpu.sync_copy(ra, acc.at[ck.at[b]], add=True)
                return 0
            lax.fori_loop(0, nb, dr, 0)

        def one_pass(p, _):
            lo = (2 * p + cid) * RCH

            def zslice(z, _):
                pltpu.sync_copy(zbuf,
                                acc.at[pl.ds(sid * (RCH // 16) + z * 8, 8)])
                return 0
            lax.fori_loop(0, RCH // 128, zslice, 0)

            @pl.when(sid == 0)
            def _():
                pltpu.sync_copy(zbuf, acc.at[pl.ds(RG, 8)])
            plsc.subcore_barrier()

            def win(wi, fill):
                base = sid * (2 * LW) + wi * WS
                pltpu.sync_copy(ia_h.at[pl.ds(base, WS)], iaw)
                pltpu.sync_copy(ib_h.at[pl.ds(base, WS)], ibw)
                pltpu.sync_copy(key_h.at[pl.ds(base, WS)], keyw)

                def step(st, fill):
                    i0 = st * 16
                    kv = keyw[pl.ds(i0, 16)]
                    m = (kv >= lo) & (kv < lo + RCH)
                    cnt = plsc.all_reduce_population_count(m)[0]

                    @pl.when(cnt > 0)
                    def _():
                        pos = fill + prefix(jnp.where(m, 1, 0)) - 1
                        pos_hi = pos >> 6
                        pos_lo = pos & (BB - 1)
                        plsc.store_scatter(ca, [pos_hi, pos_lo],
                                           iaw[pl.ds(i0, 16)], mask=m)
                        plsc.store_scatter(cb, [pos_hi, pos_lo],
                                           ibw[pl.ds(i0, 16)], mask=m)
                        plsc.store_scatter(ck, [pos_hi, pos_lo], kv - lo,
                                           mask=m)
                    fill = fill + cnt

                    def fire(f):
                        pad_tail(f)
                        drain_all(f)
                        return 0
                    return lax.cond(fill >= CB - 16, fire, lambda f: f, fill)
                return lax.fori_loop(0, WS // 16, step, fill)
            fill = lax.fori_loop(0, 2 * LW // WS, win, 0)
            pad_tail(fill)
            drain_all(fill)
            plsc.subcore_barrier()
            rows = RCH // 16
            pltpu.sync_copy(acc.at[pl.ds(sid * rows, rows)],
                            s_out.at[pl.ds(lo + sid * rows, rows)])
            plsc.subcore_barrier()
            return 0
        lax.fori_loop(0, NPASS, one_pass, 0)
    return k(ta, tb, ia, ib, key, bias)


def _inv_add(x, inv):
    @functools.partial(
        pl.kernel, mesh=_mesh,
        compiler_params=pltpu.CompilerParams(needs_layout_passes=False),
        out_type=jax.ShapeDtypeStruct((E, D), jnp.float32),
        scratch_types=[
            pltpu.VMEM((_WC,), jnp.int32),
            pltpu.VMEM((_WC, D), jnp.float32),
            pltpu.VMEM((_WC, D), jnp.float32),
            pltpu.SemaphoreType.DMA,
        ],
    )
    def k(x_h, inv_h, y_out, iv, rx, ri, sem):
        cid = lax.axis_index("c")
        sid = lax.axis_index("s")
        wid = sid * NC + cid

        def win(wi, _):
            base = wid * (E // NW) + wi * _WC
            pltpu.sync_copy(inv_h.at[pl.ds(base, _WC)], iv)
            ca = pltpu.async_copy(x_h.at[iv], ri, sem)
            pltpu.sync_copy(x_h.at[pl.ds(base, _WC)], rx)
            ca.wait()

            def rowfn(r, _):
                for cc in range(8):
                    sl = pl.ds(cc * 16, 16)
                    rx[r, sl] = rx[r, sl] + ri[r, sl]
                return 0
            lax.fori_loop(0, _WC, rowfn, 0)
            pltpu.sync_copy(rx, y_out.at[pl.ds(base, _WC)])
            return 0
        lax.fori_loop(0, _NWIN_C, win, 0)
    return k(x, inv)


def _pad_tri(t):
    pad = TP - T
    ia = jnp.concatenate([t[1], jnp.zeros((pad,), jnp.int32)])
    ib = jnp.concatenate([t[2], jnp.zeros((pad,), jnp.int32)])
    key = jnp.concatenate([t[0], jnp.full((pad,), KEY_SENTINEL, jnp.int32)])
    return ia, ib, key


def kernel(edge_attrs_0, edge_attrs_1, edge_attrs_2, edge_index_1,
           edge_index_2, tri_111, tri_222, tri_112, tri_221,
           inverse_edges_1, inverse_edges_2, params, eps):
    p = params
    W = lambda n: p[n][0]
    b = lambda n: p[n][1]

    p01, p02 = _proj(edge_attrs_0, [W("lin0j_1"), W("lin0j_2")],
                     jnp.zeros((D,), jnp.float32), False, 400)
    q1, b111, b112, l12, r21 = _proj(
        edge_attrs_1,
        [W("linj0_1"), W("linjjj_1"), W("liniij_12"), W("linijisl_12"),
         W("linijisr_21")], b("linj0_1"), True, 512)
    q2, b222, b221, l21, r12 = _proj(
        edge_attrs_2,
        [W("linj0_2"), W("linjjj_2"), W("liniij_21"), W("linijisl_21"),
         W("linijisr_12")], b("linj0_2"), True, 512)

    s1i = edge_index_1[0]
    e1i = edge_index_1[1]
    s2i = edge_index_2[0]
    e2i = edge_index_2[1]
    h1, g1 = _edge_stage(p01, s1i, e1i, q1, b("lin0j_1"))
    h2, g2 = _edge_stage(p02, s2i, e2i, q2, b("lin0j_2"))

    ia, ib_, key = _pad_tri(tri_111)
    s111 = _tri_segsum(b111, b111, ia, ib_, key, b("linjjj_1"))
    ia, ib_, key = _pad_tri(tri_222)
    s222 = _tri_segsum(b222, b222, ia, ib_, key, b("linjjj_2"))
    t = tri_112
    pad = TP - T
    i0p = jnp.concatenate([t[0], jnp.zeros((pad,), jnp.int32)])
    i1p = jnp.concatenate([t[1], jnp.zeros((pad,), jnp.int32)])
    i2p = jnp.concatenate([t[2], jnp.zeros((pad,), jnp.int32)])
    k0s = jnp.concatenate([t[0], jnp.full((pad,), KEY_SENTINEL, jnp.int32)])
    k2s = jnp.concatenate([t[2], jnp.full((pad,), KEY_SENTINEL, jnp.int32)])
    s3 = _tri_segsum(b112, b112, i0p, i1p, k2s, b("liniij_12"))
    s4 = _tri_segsum(l12, r12, i1p, i2p, k0s,
                     b("linijisl_12") + b("linijisr_12"))
    t = tri_221
    i0p = jnp.concatenate([t[0], jnp.zeros((pad,), jnp.int32)])
    i1p = jnp.concatenate([t[1], jnp.zeros((pad,), jnp.int32)])
    i2p = jnp.concatenate([t[2], jnp.zeros((pad,), jnp.int32)])
    k0s = jnp.concatenate([t[0], jnp.full((pad,), KEY_SENTINEL, jnp.int32)])
    k2s = jnp.concatenate([t[2], jnp.full((pad,), KEY_SENTINEL, jnp.int32)])
    s5 = _tri_segsum(b221, b221, i0p, i1p, k2s, b("liniij_21"))
    s6 = _tri_segsum(l21, r21, i1p, i2p, k0s,
                     b("linijisl_21") + b("linijisr_21"))

    s4p = _inv_add(s4[:E], inverse_edges_1)
    s6p = _inv_add(s6[:E], inverse_edges_2)

    scale_row = jnp.broadcast_to((1.0 + eps).astype(jnp.float32), (1, D))
    o0 = _final0(edge_attrs_0, g1, g2, p["lins_110"], p["lins_220"],
                 p["mlp0_a"], p["mlp0_b"], scale_row, 400)
    o1 = _final12(edge_attrs_1, [h1, s111[:E], s4p, s5[:E]],
                  [p["lins_011"], p["lins_111"], p["lins_121"],
                   p["lins_122"]], p["mlp1_a"], p["mlp1_b"], scale_row, 512)
    o2 = _final12(edge_attrs_2, [h2, s222[:E], s3[:E], s6p[:E]],
                  [p["lins_022"], p["lins_222"], p["lins_211"],
                   p["lins_212"]], p["mlp2_a"], p["mlp2_b"], scale_row, 512)
    return (o0, o1, o2)
